# Initial kernel scaffold; baseline (speedup 1.0000x reference)
import jax
import jax.numpy as jnp
from jax.experimental import pallas as pl


def kernel(emojis, table):
    raise NotImplementedError("write your pallas kernel here")


# speedup vs baseline: 1.1001x; 1.1001x over previous
import functools

import jax
import jax.numpy as jnp
from jax import lax
from jax.experimental import pallas as pl
from jax.experimental.pallas import tpu as pltpu
from jax.experimental.pallas import tpu_sc as plsc


@functools.cache
def _make_gather(V, D, B):
    info = plsc.get_sparse_core_info()
    NC, NS = info.num_cores, info.num_subcores
    NW = NC * NS
    assert B % NW == 0
    b_per_w = B // NW
    CH = 1280
    assert b_per_w % CH == 0
    n_chunks = b_per_w // CH
    mesh = plsc.VectorSubcoreMesh(core_axis_name="c", subcore_axis_name="s")

    @functools.partial(
        pl.kernel,
        mesh=mesh,
        compiler_params=pltpu.CompilerParams(use_tc_tiling_on_sc=False),
        out_type=jax.ShapeDtypeStruct((B, D), jnp.float32),
        scratch_types=[
            pltpu.VMEM((CH,), jnp.int32),
            pltpu.VMEM((CH, D), jnp.float32),
            pltpu.SemaphoreType.DMA,
        ],
    )
    def k(table_hbm, idx_hbm, out_hbm, idx_v, rows_v, sem):
        wid = lax.axis_index("s") * NC + lax.axis_index("c")
        base = wid * b_per_w

        def body(i, carry):
            off = base + i * CH
            pltpu.sync_copy(idx_hbm.at[pl.ds(off, CH)], idx_v)
            pltpu.async_copy(table_hbm.at[idx_v], rows_v, sem).wait()
            pltpu.sync_copy(rows_v, out_hbm.at[pl.ds(off, CH)])
            return carry

        lax.fori_loop(0, n_chunks, body, 0)

    return k


def kernel(emojis, table):
    Bq, S = emojis.shape
    V, D = table.shape
    idx = emojis.reshape(-1)
    gathered = _make_gather(V, D, idx.shape[0])(table, idx)
    return gathered.reshape(Bq, S, D)


# speedup vs baseline: 1.1141x; 1.0127x over previous
import functools

import jax
import jax.numpy as jnp
from jax import lax
from jax.experimental import pallas as pl
from jax.experimental.pallas import tpu as pltpu
from jax.experimental.pallas import tpu_sc as plsc


@functools.cache
def _make_gather(V, D, B):
    info = plsc.get_sparse_core_info()
    NC, NS = info.num_cores, info.num_subcores
    NW = NC * NS
    assert B % NW == 0
    b_per_w = B // NW
    CH = 1600
    assert b_per_w % CH == 0
    n_chunks = b_per_w // CH
    mesh = plsc.VectorSubcoreMesh(core_axis_name="c", subcore_axis_name="s")

    @functools.partial(
        pl.kernel,
        mesh=mesh,
        compiler_params=pltpu.CompilerParams(use_tc_tiling_on_sc=False),
        out_type=jax.ShapeDtypeStruct((B, D), jnp.float32),
        scratch_types=[
            pltpu.VMEM((b_per_w,), jnp.int32),
            pltpu.VMEM((2, CH, D), jnp.float32),
            pltpu.SemaphoreType.DMA,
            pltpu.SemaphoreType.DMA,
        ],
    )
    def k(table_hbm, idx_hbm, out_hbm, idx_v, rows_v, sem_g, sem_w):
        wid = lax.axis_index("s") * NC + lax.axis_index("c")
        base = wid * b_per_w

        pltpu.sync_copy(idx_hbm.at[pl.ds(base, b_per_w)], idx_v)

        def start_gather(g):
            return pltpu.async_copy(
                table_hbm.at[idx_v.at[pl.ds(g * CH, CH)]],
                rows_v.at[g % 2], sem_g)

        def start_write(g):
            return pltpu.async_copy(
                rows_v.at[g % 2],
                out_hbm.at[pl.ds(base + g * CH, CH)], sem_w)

        gathers = [None] * n_chunks
        writes = [None] * n_chunks
        gathers[0] = start_gather(0)
        for g in range(n_chunks):
            if g + 1 < n_chunks:
                if g >= 1:
                    writes[g - 1].wait()
                gathers[g + 1] = start_gather(g + 1)
            gathers[g].wait()
            writes[g] = start_write(g)
        writes[n_chunks - 2].wait()
        writes[n_chunks - 1].wait()

    return k


def kernel(emojis, table):
    Bq, S = emojis.shape
    V, D = table.shape
    idx = emojis.reshape(-1)
    gathered = _make_gather(V, D, idx.shape[0])(table, idx)
    return gathered.reshape(Bq, S, D)
